# baseline (speedup 1.0000x reference)
import functools

import jax
import jax.numpy as jnp
import numpy as np
from jax import lax
from jax.experimental import pallas as pl
from jax.experimental.pallas import tpu as pltpu
from jax.experimental.pallas import tpu_sc as plsc

L = 8
T = 1 << 16
P1 = np.int32(np.uint32(2654435761))
P2 = np.int32(805459861)
NWORKERS = 32
NCHUNKS = NWORKERS // L
STRIPE = 2048
LANES = 16


def _sc_encode(n):
    chunk = n // NCHUNKS
    nstripes = chunk // STRIPE
    groups = STRIPE // LANES
    mesh = plsc.VectorSubcoreMesh(core_axis_name="c", subcore_axis_name="s")

    @functools.partial(
        pl.kernel,
        out_type=jax.ShapeDtypeStruct((2 * L * n,), jnp.float32),
        mesh=mesh,
        scratch_types=[
            pltpu.VMEM((T,), jnp.int32),
            pltpu.VMEM((6 * STRIPE,), jnp.float32),
            pltpu.VMEM((2 * STRIPE,), jnp.float32),
            pltpu.VMEM((2 * STRIPE,), jnp.float32),
            pltpu.SemaphoreType.DMA,
            pltpu.SemaphoreType.DMA,
            pltpu.SemaphoreType.DMA,
            pltpu.SemaphoreType.DMA,
        ],
        compiler_params=pltpu.CompilerParams(use_tc_tiling_on_sc=False,
                                             needs_layout_passes=False),
    )
    def encode(uvi_f, ptab, out, tab_v, uv_v, e0_v, e1_v,
               sin_a, sin_b, sout_a, sout_b):
        wid = lax.axis_index("s") * 2 + lax.axis_index("c")
        level = wid % L
        chunk_id = wid // L
        res_i = lax.shift_left(np.int32(16), level)
        res_f = res_i.astype(jnp.float32)
        rp1 = res_i + 1
        is_dense = level < 2

        pltpu.sync_copy(ptab.at[pl.ds(level * T, T)], tab_v)

        def in_copies(s, uvoff, sem):
            base = chunk_id * chunk + jnp.minimum(s, nstripes - 1) * STRIPE
            for plane in range(3):
                yield (uvi_f.at[pl.ds(plane * n + base, STRIPE)],
                       uv_v.at[pl.ds(uvoff + plane * STRIPE, STRIPE)], sem)

        def out_copies(s, eoff, sem):
            base = chunk_id * chunk + s * STRIPE
            yield (e0_v.at[pl.ds(eoff, STRIPE)],
                   out.at[pl.ds(2 * level * n + base, STRIPE)], sem)
            yield (e1_v.at[pl.ds(eoff, STRIPE)],
                   out.at[pl.ds((2 * level + 1) * n + base, STRIPE)], sem)

        def start(copies):
            for src, dst, sem in copies:
                pltpu.async_copy(src, dst, sem)

        def wait(copies):
            for src, dst, sem in copies:
                pltpu.make_async_copy(src, dst, sem).wait()

        def make_pass(dense):
            def compute_stripe(uvoff, eoff):
                def group_body(g, _):
                    x = uv_v[pl.ds(uvoff + g * LANES, LANES)] * res_f
                    y = uv_v[pl.ds(uvoff + STRIPE + g * LANES, LANES)] * res_f
                    z = (uv_v[pl.ds(uvoff + 2 * STRIPE + g * LANES, LANES)]
                         * res_f)
                    ix = x.astype(jnp.int32)
                    iy = y.astype(jnp.int32)
                    iz = z.astype(jnp.int32)
                    wx = x - ix.astype(jnp.float32)
                    wy = y - iy.astype(jnp.float32)
                    wz = z - iz.astype(jnp.float32)
                    ixc = jnp.minimum(ix + 1, res_i)
                    iyc = jnp.minimum(iy + 1, res_i)
                    izc = jnp.minimum(iz + 1, res_i)
                    px = (ix, ixc)
                    if dense:
                        yz = tuple(rp1 * (py + rp1 * pz)
                                   for pz in (iz, izc) for py in (iy, iyc))
                    else:
                        hy = (iy * P1, iyc * P1)
                        hz = (iz * P2, izc * P2)
                    sx = (1.0 - wx, wx)
                    sxy = tuple(sy * s for s in (1.0 - wy, wy) for sy in sx)
                    szw = (1.0 - wz, wz)
                    acc0 = jnp.zeros((LANES,), jnp.float32)
                    acc1 = jnp.zeros((LANES,), jnp.float32)
                    for corner in range(8):
                        ox = corner & 1
                        oy = (corner >> 1) & 1
                        oz = (corner >> 2) & 1
                        if dense:
                            idx = px[ox] + yz[oz * 2 + oy]
                        else:
                            idx = ((px[ox] ^ hy[oy] ^ hz[oz])
                                   & np.int32(T - 1))
                        word = plsc.load_gather(tab_v, [idx])
                        f0 = lax.bitcast_convert_type(word << 16, jnp.float32)
                        f1 = lax.bitcast_convert_type(word & np.int32(-65536),
                                                      jnp.float32)
                        wc = sxy[oy * 2 + ox] * szw[oz]
                        acc0 = acc0 + wc * f0
                        acc1 = acc1 + wc * f1
                    sl = pl.ds(eoff + g * LANES, LANES)
                    e0_v[sl] = acc0
                    e1_v[sl] = acc1
                    return ()

                lax.fori_loop(0, groups, group_body, (), unroll=2)

            def pair_body(k, _):
                s0 = 2 * k
                s1 = 2 * k + 1
                wait(in_copies(s0, 0, sin_a))
                compute_stripe(0, 0)
                start(in_copies(s0 + 2, 0, sin_a))
                start(out_copies(s0, 0, sout_a))
                wait(in_copies(s1, 3 * STRIPE, sin_b))
                compute_stripe(3 * STRIPE, STRIPE)
                start(in_copies(s1 + 2, 3 * STRIPE, sin_b))
                start(out_copies(s1, STRIPE, sout_b))
                wait(out_copies(s0, 0, sout_a))
                wait(out_copies(s1, STRIPE, sout_b))
                return ()

            def run():
                start(in_copies(np.int32(0), 0, sin_a))
                start(in_copies(np.int32(1), 3 * STRIPE, sin_b))
                lax.fori_loop(0, nstripes // 2, pair_body, (), unroll=False)
                wait(in_copies(np.int32(0), 0, sin_a))
                wait(in_copies(np.int32(0), 3 * STRIPE, sin_b))

            return run

        lax.cond(is_dense, make_pass(True), make_pass(False))

    return encode


def _mlp(enc_t, w1t, b1c, w2t, b2c, n, bn=16384):
    def body(e_ref, w1_ref, b1_ref, w2_ref, b2_ref, o_ref):
        e = e_ref[...]
        h = jnp.dot(w1_ref[...], e, preferred_element_type=jnp.float32)
        h = jnp.maximum(h + b1_ref[...], 0.0)
        zz = jnp.dot(w2_ref[...], h, preferred_element_type=jnp.float32)
        zz = zz + b2_ref[...]
        o_ref[...] = 1.0 / (1.0 + jnp.exp(-zz))

    return pl.pallas_call(
        body,
        grid=(n // bn,),
        in_specs=[
            pl.BlockSpec((2 * L, bn), lambda i: (0, i)),
            pl.BlockSpec((64, 2 * L), lambda i: (0, 0)),
            pl.BlockSpec((64, 1), lambda i: (0, 0)),
            pl.BlockSpec((1, 64), lambda i: (0, 0)),
            pl.BlockSpec((1, 1), lambda i: (0, 0)),
        ],
        out_specs=pl.BlockSpec((1, bn), lambda i: (0, i)),
        out_shape=jax.ShapeDtypeStruct((1, n), jnp.float32),
    )(enc_t, w1t, b1c, w2t, b2c)


def kernel(uvi, tables, W1, b1, W2, b2):
    n = uvi.shape[0]
    t16 = tables.astype(jnp.bfloat16)
    bits = lax.bitcast_convert_type(t16, jnp.uint16).astype(jnp.uint32)
    ptab = (bits[..., 0] | (bits[..., 1] << 16)).astype(jnp.int32)
    uvi_f = uvi.T.reshape(3 * n)

    enc_t = _sc_encode(n)(uvi_f, ptab.reshape(L * T)).reshape(2 * L, n)

    mask_t = _mlp(enc_t, W1.T, b1.reshape(64, 1), W2.T, b2.reshape(1, 1), n)
    return mask_t.reshape(n, 1)


# speedup vs baseline: 1.1273x; 1.1273x over previous
import functools

import jax
import jax.numpy as jnp
import numpy as np
from jax import lax
from jax.experimental import pallas as pl
from jax.experimental.pallas import tpu as pltpu
from jax.experimental.pallas import tpu_sc as plsc

L = 8
T = 1 << 16
P1 = np.int32(np.uint32(2654435761))
P2 = np.int32(805459861)
NWORKERS = 32
NCHUNKS = NWORKERS // L
STRIPE = 2048
LANES = 16


def _sc_encode(n, half, nh):
    chunk = nh // NCHUNKS
    nstripes = chunk // STRIPE
    groups = STRIPE // LANES
    mesh = plsc.VectorSubcoreMesh(core_axis_name="c", subcore_axis_name="s")

    @functools.partial(
        pl.kernel,
        out_type=jax.ShapeDtypeStruct((2 * L * nh,), jnp.float32),
        mesh=mesh,
        scratch_types=[
            pltpu.VMEM((T,), jnp.int32),
            pltpu.VMEM((6 * STRIPE,), jnp.float32),
            pltpu.VMEM((2 * STRIPE,), jnp.float32),
            pltpu.VMEM((2 * STRIPE,), jnp.float32),
            pltpu.SemaphoreType.DMA,
            pltpu.SemaphoreType.DMA,
            pltpu.SemaphoreType.DMA,
            pltpu.SemaphoreType.DMA,
        ],
        compiler_params=pltpu.CompilerParams(use_tc_tiling_on_sc=False,
                                             needs_layout_passes=False),
    )
    def encode(uvi_f, ptab, out, tab_v, uv_v, e0_v, e1_v,
               sin_a, sin_b, sout_a, sout_b):
        wid = lax.axis_index("s") * 2 + lax.axis_index("c")
        level = wid % L
        chunk_id = wid // L
        res_i = lax.shift_left(np.int32(16), level)
        res_f = res_i.astype(jnp.float32)
        rp1 = res_i + 1
        is_dense = level < 2

        pltpu.sync_copy(ptab.at[pl.ds(level * T, T)], tab_v)

        def in_copies(s, uvoff, sem):
            base = (half * nh + chunk_id * chunk
                    + jnp.minimum(s, nstripes - 1) * STRIPE)
            for plane in range(3):
                yield (uvi_f.at[pl.ds(plane * n + base, STRIPE)],
                       uv_v.at[pl.ds(uvoff + plane * STRIPE, STRIPE)], sem)

        def out_copies(s, eoff, sem):
            base = chunk_id * chunk + s * STRIPE
            yield (e0_v.at[pl.ds(eoff, STRIPE)],
                   out.at[pl.ds(2 * level * nh + base, STRIPE)], sem)
            yield (e1_v.at[pl.ds(eoff, STRIPE)],
                   out.at[pl.ds((2 * level + 1) * nh + base, STRIPE)], sem)

        def start(copies):
            for src, dst, sem in copies:
                pltpu.async_copy(src, dst, sem)

        def wait(copies):
            for src, dst, sem in copies:
                pltpu.make_async_copy(src, dst, sem).wait()

        def make_pass(dense):
            def compute_stripe(uvoff, eoff):
                def group_body(g, _):
                    x = uv_v[pl.ds(uvoff + g * LANES, LANES)] * res_f
                    y = uv_v[pl.ds(uvoff + STRIPE + g * LANES, LANES)] * res_f
                    z = (uv_v[pl.ds(uvoff + 2 * STRIPE + g * LANES, LANES)]
                         * res_f)
                    ix = x.astype(jnp.int32)
                    iy = y.astype(jnp.int32)
                    iz = z.astype(jnp.int32)
                    wx = x - ix.astype(jnp.float32)
                    wy = y - iy.astype(jnp.float32)
                    wz = z - iz.astype(jnp.float32)
                    ixc = jnp.minimum(ix + 1, res_i)
                    iyc = jnp.minimum(iy + 1, res_i)
                    izc = jnp.minimum(iz + 1, res_i)
                    px = (ix, ixc)
                    if dense:
                        yz = tuple(rp1 * (py + rp1 * pz)
                                   for pz in (iz, izc) for py in (iy, iyc))
                    else:
                        hy = (iy * P1, iyc * P1)
                        hz = (iz * P2, izc * P2)
                    sx = (1.0 - wx, wx)
                    sxy = tuple(sy * s for s in (1.0 - wy, wy) for sy in sx)
                    szw = (1.0 - wz, wz)
                    acc0 = jnp.zeros((LANES,), jnp.float32)
                    acc1 = jnp.zeros((LANES,), jnp.float32)
                    for corner in range(8):
                        ox = corner & 1
                        oy = (corner >> 1) & 1
                        oz = (corner >> 2) & 1
                        if dense:
                            idx = px[ox] + yz[oz * 2 + oy]
                        else:
                            idx = ((px[ox] ^ hy[oy] ^ hz[oz])
                                   & np.int32(T - 1))
                        word = plsc.load_gather(tab_v, [idx])
                        f0 = lax.bitcast_convert_type(word << 16, jnp.float32)
                        f1 = lax.bitcast_convert_type(word & np.int32(-65536),
                                                      jnp.float32)
                        wc = sxy[oy * 2 + ox] * szw[oz]
                        acc0 = acc0 + wc * f0
                        acc1 = acc1 + wc * f1
                    sl = pl.ds(eoff + g * LANES, LANES)
                    e0_v[sl] = acc0
                    e1_v[sl] = acc1
                    return ()

                lax.fori_loop(0, groups, group_body, (), unroll=False)

            def pair_body(k, _):
                s0 = 2 * k
                s1 = 2 * k + 1
                wait(in_copies(s0, 0, sin_a))
                compute_stripe(0, 0)
                start(in_copies(s0 + 2, 0, sin_a))
                start(out_copies(s0, 0, sout_a))
                wait(in_copies(s1, 3 * STRIPE, sin_b))
                compute_stripe(3 * STRIPE, STRIPE)
                start(in_copies(s1 + 2, 3 * STRIPE, sin_b))
                start(out_copies(s1, STRIPE, sout_b))
                wait(out_copies(s0, 0, sout_a))
                wait(out_copies(s1, STRIPE, sout_b))
                return ()

            def run():
                start(in_copies(np.int32(0), 0, sin_a))
                start(in_copies(np.int32(1), 3 * STRIPE, sin_b))
                lax.fori_loop(0, nstripes // 2, pair_body, (), unroll=False)
                wait(in_copies(np.int32(0), 0, sin_a))
                wait(in_copies(np.int32(0), 3 * STRIPE, sin_b))

            return run

        lax.cond(is_dense, make_pass(True), make_pass(False))

    return encode


def _mlp(enc_t, w1t, b1c, w2t, b2c, n, bn=16384):
    def body(e_ref, w1_ref, b1_ref, w2_ref, b2_ref, o_ref):
        e = e_ref[...]
        h = jnp.dot(w1_ref[...], e, preferred_element_type=jnp.float32)
        h = jnp.maximum(h + b1_ref[...], 0.0)
        zz = jnp.dot(w2_ref[...], h, preferred_element_type=jnp.float32)
        zz = zz + b2_ref[...]
        o_ref[...] = 1.0 / (1.0 + jnp.exp(-zz))

    return pl.pallas_call(
        body,
        grid=(n // bn,),
        in_specs=[
            pl.BlockSpec((2 * L, bn), lambda i: (0, i)),
            pl.BlockSpec((64, 2 * L), lambda i: (0, 0)),
            pl.BlockSpec((64, 1), lambda i: (0, 0)),
            pl.BlockSpec((1, 64), lambda i: (0, 0)),
            pl.BlockSpec((1, 1), lambda i: (0, 0)),
        ],
        out_specs=pl.BlockSpec((1, bn), lambda i: (0, i)),
        out_shape=jax.ShapeDtypeStruct((1, n), jnp.float32),
    )(enc_t, w1t, b1c, w2t, b2c)


def kernel(uvi, tables, W1, b1, W2, b2):
    n = uvi.shape[0]
    t16 = tables.astype(jnp.bfloat16)
    bits = lax.bitcast_convert_type(t16, jnp.uint16).astype(jnp.uint32)
    ptab = (bits[..., 0] | (bits[..., 1] << 16)).astype(jnp.int32)
    uvi_f = uvi.T.reshape(3 * n)
    ptab_f = ptab.reshape(L * T)
    w1t = W1.T
    b1c = b1.reshape(64, 1)
    w2t = W2.T
    b2c = b2.reshape(1, 1)

    nh = n // 2
    enc0 = _sc_encode(n, 0, nh)(uvi_f, ptab_f).reshape(2 * L, nh)
    enc1 = _sc_encode(n, 1, nh)(uvi_f, ptab_f).reshape(2 * L, nh)
    mask0 = _mlp(enc0, w1t, b1c, w2t, b2c, nh)
    mask1 = _mlp(enc1, w1t, b1c, w2t, b2c, nh)
    return jnp.concatenate([mask0, mask1], axis=1).reshape(n, 1)


# speedup vs baseline: 1.2022x; 1.0664x over previous
import functools

import jax
import jax.numpy as jnp
import numpy as np
from jax import lax
from jax.experimental import pallas as pl
from jax.experimental.pallas import tpu as pltpu
from jax.experimental.pallas import tpu_sc as plsc

L = 8
T = 1 << 16
P1 = np.int32(np.uint32(2654435761))
P2 = np.int32(805459861)
NWORKERS = 32
NCHUNKS = NWORKERS // L
STRIPE = 4096
LANES = 16


def _sc_encode(n, half, nh):
    chunk = nh // NCHUNKS
    nstripes = chunk // STRIPE
    groups = STRIPE // LANES
    mesh = plsc.VectorSubcoreMesh(core_axis_name="c", subcore_axis_name="s")

    @functools.partial(
        pl.kernel,
        out_type=jax.ShapeDtypeStruct((2 * L * nh,), jnp.float32),
        mesh=mesh,
        scratch_types=[
            pltpu.VMEM((T,), jnp.int32),
            pltpu.VMEM((6 * STRIPE,), jnp.float32),
            pltpu.VMEM((2 * STRIPE,), jnp.float32),
            pltpu.VMEM((2 * STRIPE,), jnp.float32),
            pltpu.SemaphoreType.DMA,
            pltpu.SemaphoreType.DMA,
            pltpu.SemaphoreType.DMA,
            pltpu.SemaphoreType.DMA,
        ],
        compiler_params=pltpu.CompilerParams(use_tc_tiling_on_sc=False,
                                             needs_layout_passes=False),
    )
    def encode(uvi_f, ptab, out, tab_v, uv_v, e0_v, e1_v,
               sin_a, sin_b, sout_a, sout_b):
        wid = lax.axis_index("s") * 2 + lax.axis_index("c")
        level = wid % L
        chunk_id = wid // L
        res_i = lax.shift_left(np.int32(16), level)
        res_f = res_i.astype(jnp.float32)
        rp1 = res_i + 1
        is_dense = level < 2

        pltpu.sync_copy(ptab.at[pl.ds(level * T, T)], tab_v)

        def in_copies(s, uvoff, sem):
            base = (half * nh + chunk_id * chunk
                    + jnp.minimum(s, nstripes - 1) * STRIPE)
            for plane in range(3):
                yield (uvi_f.at[pl.ds(plane * n + base, STRIPE)],
                       uv_v.at[pl.ds(uvoff + plane * STRIPE, STRIPE)], sem)

        def out_copies(s, eoff, sem):
            base = chunk_id * chunk + s * STRIPE
            yield (e0_v.at[pl.ds(eoff, STRIPE)],
                   out.at[pl.ds(2 * level * nh + base, STRIPE)], sem)
            yield (e1_v.at[pl.ds(eoff, STRIPE)],
                   out.at[pl.ds((2 * level + 1) * nh + base, STRIPE)], sem)

        def start(copies):
            for src, dst, sem in copies:
                pltpu.async_copy(src, dst, sem)

        def wait(copies):
            for src, dst, sem in copies:
                pltpu.make_async_copy(src, dst, sem).wait()

        def make_pass(dense):
            def compute_stripe(uvoff, eoff):
                def group_body(g, _):
                    x = uv_v[pl.ds(uvoff + g * LANES, LANES)] * res_f
                    y = uv_v[pl.ds(uvoff + STRIPE + g * LANES, LANES)] * res_f
                    z = (uv_v[pl.ds(uvoff + 2 * STRIPE + g * LANES, LANES)]
                         * res_f)
                    ix = x.astype(jnp.int32)
                    iy = y.astype(jnp.int32)
                    iz = z.astype(jnp.int32)
                    wx = x - ix.astype(jnp.float32)
                    wy = y - iy.astype(jnp.float32)
                    wz = z - iz.astype(jnp.float32)
                    ixc = jnp.minimum(ix + 1, res_i)
                    iyc = jnp.minimum(iy + 1, res_i)
                    izc = jnp.minimum(iz + 1, res_i)
                    px = (ix, ixc)
                    if dense:
                        yz = tuple(rp1 * (py + rp1 * pz)
                                   for pz in (iz, izc) for py in (iy, iyc))
                    else:
                        hy = (iy * P1, iyc * P1)
                        hz = (iz * P2, izc * P2)
                        yz = tuple(hz[b] ^ hy[a]
                                   for b in range(2) for a in range(2))
                    sx = (1.0 - wx, wx)
                    sxy = tuple(sy * s for s in (1.0 - wy, wy) for sy in sx)
                    szw = (1.0 - wz, wz)
                    acc0 = jnp.zeros((LANES,), jnp.float32)
                    acc1 = jnp.zeros((LANES,), jnp.float32)
                    for corner in range(8):
                        ox = corner & 1
                        oy = (corner >> 1) & 1
                        oz = (corner >> 2) & 1
                        if dense:
                            idx = px[ox] + yz[oz * 2 + oy]
                        else:
                            idx = (px[ox] ^ yz[oz * 2 + oy]) & np.int32(T - 1)
                        word = plsc.load_gather(tab_v, [idx])
                        f0 = lax.bitcast_convert_type(word << 16, jnp.float32)
                        f1 = lax.bitcast_convert_type(word & np.int32(-65536),
                                                      jnp.float32)
                        wc = sxy[oy * 2 + ox] * szw[oz]
                        acc0 = acc0 + wc * f0
                        acc1 = acc1 + wc * f1
                    sl = pl.ds(eoff + g * LANES, LANES)
                    e0_v[sl] = acc0
                    e1_v[sl] = acc1
                    return ()

                lax.fori_loop(0, groups, group_body, (), unroll=False)

            def pair_body(k, _):
                s0 = 2 * k
                s1 = 2 * k + 1
                wait(in_copies(s0, 0, sin_a))
                compute_stripe(0, 0)
                start(in_copies(s0 + 2, 0, sin_a))
                start(out_copies(s0, 0, sout_a))
                wait(in_copies(s1, 3 * STRIPE, sin_b))
                compute_stripe(3 * STRIPE, STRIPE)
                start(in_copies(s1 + 2, 3 * STRIPE, sin_b))
                start(out_copies(s1, STRIPE, sout_b))
                wait(out_copies(s0, 0, sout_a))
                wait(out_copies(s1, STRIPE, sout_b))
                return ()

            def run():
                start(in_copies(np.int32(0), 0, sin_a))
                start(in_copies(np.int32(1), 3 * STRIPE, sin_b))
                lax.fori_loop(0, nstripes // 2, pair_body, (), unroll=False)
                wait(in_copies(np.int32(0), 0, sin_a))
                wait(in_copies(np.int32(0), 3 * STRIPE, sin_b))

            return run

        lax.cond(is_dense, make_pass(True), make_pass(False))

    return encode


def _mlp(enc_t, w1t, b1c, w2t, b2c, n, bn=32768):
    def body(e_ref, w1_ref, b1_ref, w2_ref, b2_ref, o_ref):
        e = e_ref[...]
        h = jnp.dot(w1_ref[...], e, preferred_element_type=jnp.float32)
        h = jnp.maximum(h + b1_ref[...], 0.0)
        zz = jnp.dot(w2_ref[...], h, preferred_element_type=jnp.float32)
        zz = zz + b2_ref[...]
        o_ref[...] = 1.0 / (1.0 + jnp.exp(-zz))

    return pl.pallas_call(
        body,
        grid=(n // bn,),
        in_specs=[
            pl.BlockSpec((2 * L, bn), lambda i: (0, i)),
            pl.BlockSpec((64, 2 * L), lambda i: (0, 0)),
            pl.BlockSpec((64, 1), lambda i: (0, 0)),
            pl.BlockSpec((1, 64), lambda i: (0, 0)),
            pl.BlockSpec((1, 1), lambda i: (0, 0)),
        ],
        out_specs=pl.BlockSpec((1, bn), lambda i: (0, i)),
        out_shape=jax.ShapeDtypeStruct((1, n), jnp.float32),
    )(enc_t, w1t, b1c, w2t, b2c)


def kernel(uvi, tables, W1, b1, W2, b2):
    n = uvi.shape[0]
    t16 = tables.astype(jnp.bfloat16)
    bits = lax.bitcast_convert_type(t16, jnp.uint16).astype(jnp.uint32)
    ptab = (bits[..., 0] | (bits[..., 1] << 16)).astype(jnp.int32)
    uvi_f = uvi.T.reshape(3 * n)
    ptab_f = ptab.reshape(L * T)
    w1t = W1.T
    b1c = b1.reshape(64, 1)
    w2t = W2.T
    b2c = b2.reshape(1, 1)

    enc_t = _sc_encode(n, 0, n)(uvi_f, ptab_f).reshape(2 * L, n)
    mask_t = _mlp(enc_t, w1t, b1c, w2t, b2c, n)
    return mask_t.reshape(n, 1)
